# baseline (device time: 213944 ns/iter reference)
import jax
import jax.numpy as jnp
from jax import lax
from jax.experimental import pallas as pl
from jax.experimental.pallas import tpu as pltpu

N_DEV = 32
N_EXP = 64
R = N_DEV // 2
L = N_DEV - 1 - R


def kernel(x, router_W, route_idx, expert_W, shared_W):
    n_tok, d_model = x.shape
    e_per, _, d_ff = expert_W.shape

    def body(x_ref, rw_ref, idx_ref, ew_ref, sw_ref, out_ref,
             comm_r, comm_l, send_r, recv_r, send_l, recv_l,
             credit_r, credit_l):
        my = lax.axis_index("i")
        left = lax.rem(my + N_DEV - 1, N_DEV)
        right = lax.rem(my + 1, N_DEV)

        barrier_sem = pltpu.get_barrier_semaphore()
        for nbr in (left, right):
            pl.semaphore_signal(
                barrier_sem, inc=1,
                device_id=(nbr,), device_id_type=pl.DeviceIdType.MESH,
            )
        pl.semaphore_wait(barrier_sem, 2)

        xv = x_ref[:, :]
        scores = jnp.dot(xv, rw_ref[:, :], preferred_element_type=jnp.float32)
        m = jnp.max(scores, axis=1, keepdims=True)
        ex = jnp.exp(scores - m)
        probs = ex / jnp.sum(ex, axis=1, keepdims=True)
        route = idx_ref[:, :]
        e_ids = lax.broadcasted_iota(jnp.int32, (n_tok, N_EXP), 1)
        p = jnp.sum(jnp.where(route == e_ids, probs, 0.0),
                    axis=1, keepdims=True)

        acc = jnp.dot(xv, sw_ref[:, :], preferred_element_type=jnp.float32)

        ew16 = ew_ref[:, :, :].astype(jnp.bfloat16)
        comm_r[0, :, :, :] = ew16
        comm_l[0, :, :, :] = ew16

        def scaled_x(origin):
            cols = []
            for k in range(e_per):
                c = jnp.where(route == e_per * origin + k, p, 0.0)
                cols.append((c * xv).astype(jnp.bfloat16))
            return cols

        def chunk_w(comm, slot):
            return jnp.reshape(comm[slot, :, :, :], (e_per * d_model, d_ff))

        for h in range(R + 1):
            slot, nxt = h % 2, (h + 1) % 2
            if h < R:
                if h > 0:
                    pl.semaphore_wait(credit_r, 1)
                rdma_r = pltpu.make_async_remote_copy(
                    src_ref=comm_r.at[slot], dst_ref=comm_r.at[nxt],
                    send_sem=send_r.at[slot], recv_sem=recv_r.at[nxt],
                    device_id=(right,), device_id_type=pl.DeviceIdType.MESH,
                )
                rdma_r.start()
            if h < L:
                if h > 0:
                    pl.semaphore_wait(credit_l, 1)
                rdma_l = pltpu.make_async_remote_copy(
                    src_ref=comm_l.at[slot], dst_ref=comm_l.at[nxt],
                    send_sem=send_l.at[slot], recv_sem=recv_l.at[nxt],
                    device_id=(left,), device_id_type=pl.DeviceIdType.MESH,
                )
                rdma_l.start()

            if h == 0:
                cols = scaled_x(my)
                w = chunk_w(comm_r, 0)
            else:
                cols = scaled_x(lax.rem(my - h + N_DEV, N_DEV))
                w = chunk_w(comm_r, slot)
                if h <= L:
                    cols += scaled_x(lax.rem(my + h, N_DEV))
                    w = jnp.concatenate([w, chunk_w(comm_l, slot)], axis=0)
            acc = acc + jnp.dot(
                jnp.concatenate(cols, axis=1), w,
                preferred_element_type=jnp.float32,
            )

            if h < R:
                rdma_r.wait()
            if h < L:
                rdma_l.wait()
            if h < R - 1:
                pl.semaphore_signal(
                    credit_r, inc=1,
                    device_id=(left,), device_id_type=pl.DeviceIdType.MESH,
                )
            if h < L - 1:
                pl.semaphore_signal(
                    credit_l, inc=1,
                    device_id=(right,), device_id_type=pl.DeviceIdType.MESH,
                )

        out_ref[:, :] = acc

    return pl.pallas_call(
        body,
        out_shape=jax.ShapeDtypeStruct((n_tok, d_ff), jnp.float32),
        in_specs=[pl.BlockSpec(memory_space=pltpu.VMEM)] * 5,
        out_specs=pl.BlockSpec(memory_space=pltpu.VMEM),
        scratch_shapes=[
            pltpu.VMEM((2, e_per, d_model, d_ff), jnp.bfloat16),
            pltpu.VMEM((2, e_per, d_model, d_ff), jnp.bfloat16),
            pltpu.SemaphoreType.DMA((2,)),
            pltpu.SemaphoreType.DMA((2,)),
            pltpu.SemaphoreType.DMA((2,)),
            pltpu.SemaphoreType.DMA((2,)),
            pltpu.SemaphoreType.REGULAR,
            pltpu.SemaphoreType.REGULAR,
        ],
        compiler_params=pltpu.CompilerParams(collective_id=0),
    )(x, router_W, route_idx, expert_W, shared_W)
